# baseline (speedup 1.0000x reference)
import functools
import math

import jax
import jax.numpy as jnp
from jax import lax
from jax.experimental import pallas as pl
from jax.experimental.pallas import tpu as pltpu
from jax.experimental.pallas import tpu_sc as plsc

N = 10000
E = 320000
D = 128
NH = 4
HD = 32

NW = 32
CH = 80
NC = E // CH
CPB = 32
PART_NC = (992, 1024, 992, 992)

NPAD = 10240
TROWS = NPAD // 16

_INV_SQRT_HD = 1.0 / math.sqrt(HD)
_INV_SQRT2 = 1.0 / math.sqrt(2.0)


def _node_proj_body(hn_ref, w_ref, b_ref, qn_ref, kn_ref, vn_ref):
    o = jnp.dot(hn_ref[...], w_ref[...], preferred_element_type=jnp.float32)
    o = o + b_ref[...]
    qn_ref[...] = o[:, :D]
    kn_ref[...] = o[:, D:2 * D]
    vn_ref[...] = o[:, 2 * D:]


def _node_proj(h_n, w_node, b_node):
    blk = 2000
    return pl.pallas_call(
        _node_proj_body,
        grid=(N // blk,),
        in_specs=[
            pl.BlockSpec((blk, D), lambda i: (i, 0)),
            pl.BlockSpec((D, 3 * D), lambda i: (0, 0)),
            pl.BlockSpec((1, 3 * D), lambda i: (0, 0)),
        ],
        out_specs=[
            pl.BlockSpec((blk, D), lambda i: (i, 0)),
            pl.BlockSpec((blk, D), lambda i: (i, 0)),
            pl.BlockSpec((blk, D), lambda i: (i, 0)),
        ],
        out_shape=[
            jax.ShapeDtypeStruct((N, D), jnp.float32),
            jax.ShapeDtypeStruct((N, D), jnp.float32),
            jax.ShapeDtypeStruct((N, D), jnp.float32),
        ],
    )(h_n, w_node, b_node)


def _sc_gather(qn, kn, vn, src2, dst2, nchunk, hoff):
    perw = nchunk // NW
    mesh = plsc.VectorSubcoreMesh(core_axis_name="c", subcore_axis_name="s")

    @functools.partial(
        pl.kernel,
        out_type=(
            jax.ShapeDtypeStruct((nchunk, CH, D), jnp.float32),
            jax.ShapeDtypeStruct((nchunk, CH, D), jnp.float32),
            jax.ShapeDtypeStruct((nchunk, CH, D), jnp.float32),
        ),
        mesh=mesh,
        scratch_types=(
            [pltpu.VMEM((CH,), jnp.int32)] * 4
            + [pltpu.VMEM((CH, D), jnp.float32)] * 6
            + [pltpu.SemaphoreType.DMA] * 16
        ),
    )
    def k(qn_hbm, kn_hbm, vn_hbm, src_hbm, dst_hbm, qns_hbm, kns_hbm, vns_hbm,
          sxA, dxA, sxB, dxB, qbA, kbA, vbA, qbB, kbB, vbB,
          isA, idA, isB, idB,
          gqA, gkA, gvA, gqB, gkB, gvB, wqA, wkA, wvA, wqB, wkB, wvB):
        c = lax.axis_index("c")
        s = lax.axis_index("s")
        wid = s * 2 + c
        base = wid * perw

        A = (sxA, dxA, qbA, kbA, vbA, isA, idA, gqA, gkA, gvA, wqA, wkA, wvA)
        B = (sxB, dxB, qbB, kbB, vbB, isB, idB, gqB, gkB, gvB, wqB, wkB, wvB)

        def issue_ix(j, bufs):
            sx, dx = bufs[0], bufs[1]
            si, di = bufs[5], bufs[6]
            g = hoff + base + j
            pltpu.async_copy(src_hbm.at[g], sx, si)
            pltpu.async_copy(dst_hbm.at[g], dx, di)

        def wait_ix(j, bufs):
            sx, dx = bufs[0], bufs[1]
            si, di = bufs[5], bufs[6]
            g = hoff + base + j
            pltpu.make_async_copy(src_hbm.at[g], sx, si).wait()
            pltpu.make_async_copy(dst_hbm.at[g], dx, di).wait()

        def issue_g(j, bufs):
            sx, dx, qb, kb, vb = bufs[:5]
            gq, gk, gv = bufs[7:10]
            pltpu.async_copy(qn_hbm.at[sx], qb, gq)
            pltpu.async_copy(kn_hbm.at[dx], kb, gk)
            pltpu.async_copy(vn_hbm.at[dx], vb, gv)

        def wait_g(j, bufs):
            sx, dx, qb, kb, vb = bufs[:5]
            gq, gk, gv = bufs[7:10]
            pltpu.make_async_copy(qn_hbm.at[sx], qb, gq).wait()
            pltpu.make_async_copy(kn_hbm.at[dx], kb, gk).wait()
            pltpu.make_async_copy(vn_hbm.at[dx], vb, gv).wait()

        def issue_w(j, bufs):
            qb, kb, vb = bufs[2:5]
            wq, wk, wv = bufs[10:]
            chunk = base + j
            pltpu.async_copy(qb, qns_hbm.at[chunk], wq)
            pltpu.async_copy(kb, kns_hbm.at[chunk], wk)
            pltpu.async_copy(vb, vns_hbm.at[chunk], wv)

        def wait_w(j, bufs):
            qb, kb, vb = bufs[2:5]
            wq, wk, wv = bufs[10:]
            chunk = base + j
            pltpu.make_async_copy(qb, qns_hbm.at[chunk], wq).wait()
            pltpu.make_async_copy(kb, kns_hbm.at[chunk], wk).wait()
            pltpu.make_async_copy(vb, vns_hbm.at[chunk], wv).wait()

        def step(j, cur, nxt):
            @pl.when(j + 1 < perw)
            def _():
                @pl.when(j >= 1)
                def _():
                    wait_w(j - 1, nxt)

                wait_ix(j + 1, nxt)
                issue_g(j + 1, nxt)

            wait_g(j, cur)
            @pl.when(j + 2 < perw)
            def _():
                issue_ix(j + 2, cur)

            issue_w(j, cur)

        issue_ix(0, A)
        wait_ix(0, A)
        issue_g(0, A)

        @pl.when(1 < perw)
        def _():
            issue_ix(1, B)

        def body(j, carry):
            @pl.when(lax.rem(j, 2) == 0)
            def _():
                step(j, A, B)

            @pl.when(lax.rem(j, 2) == 1)
            def _():
                step(j, B, A)

            return carry

        lax.fori_loop(0, perw, body, 0)
        wait_w(perw - 1, A if (perw - 1) % 2 == 0 else B)
        wait_w(perw - 2, A if (perw - 2) % 2 == 0 else B)

    return k(qn, kn, vn, src2, dst2)


def _edge_body(he_ref, qns_ref, kns_ref, vns_ref, we_ref, m16_ref, ex_ref, pv_ref, p_ref):
    he = he_ref[...]
    qkve = jnp.dot(he, we_ref[...], preferred_element_type=jnp.float32)
    q = qkve[:, :D] + qns_ref[...]
    k = qkve[:, D:2 * D] + kns_ref[...]
    v = qkve[:, 2 * D:] + vns_ref[...]
    s16 = jnp.dot(q * k, m16_ref[...], preferred_element_type=jnp.float32)
    s16 = s16 * _INV_SQRT_HD
    col = lax.broadcasted_iota(jnp.int32, s16.shape, 1)
    p16 = jnp.where(col < NH, jnp.exp(s16), 0.0)
    pv_ref[...] = v * jnp.dot(p16, ex_ref[...], preferred_element_type=jnp.float32)
    p_ref[...] = p16


def _edge_stage(h_e, qns, kns, vns, w_edge, m16, ex16, nchunk, hblk):
    blk = CPB * CH
    ne = nchunk * CH
    return pl.pallas_call(
        _edge_body,
        grid=(ne // blk,),
        in_specs=[
            pl.BlockSpec((blk, D), lambda i: (i + hblk, 0)),
            pl.BlockSpec((blk, D), lambda i: (i, 0)),
            pl.BlockSpec((blk, D), lambda i: (i, 0)),
            pl.BlockSpec((blk, D), lambda i: (i, 0)),
            pl.BlockSpec((D, 3 * D), lambda i: (0, 0)),
            pl.BlockSpec((D, 16), lambda i: (0, 0)),
            pl.BlockSpec((16, D), lambda i: (0, 0)),
        ],
        out_specs=[
            pl.BlockSpec((blk, D), lambda i: (i, 0)),
            pl.BlockSpec((blk, 16), lambda i: (i, 0)),
        ],
        out_shape=[
            jax.ShapeDtypeStruct((ne, D), jnp.float32),
            jax.ShapeDtypeStruct((ne, 16), jnp.float32),
        ],
    )(h_e, qns, kns, vns, w_edge, m16, ex16)


def _sc_scatter(pv3, p3, src2, z128, nchunk, dbase):
    perc = nchunk // 16
    mesh = plsc.VectorSubcoreMesh(core_axis_name="c", subcore_axis_name="s")

    @functools.partial(
        pl.kernel,
        out_type=jax.ShapeDtypeStruct((2, NPAD, D), jnp.float32),
        mesh=mesh,
        scratch_types=(
            [pltpu.VMEM_SHARED((NPAD, D), jnp.float32)]
            + [pltpu.VMEM((CH, D), jnp.float32)] * 2
            + [pltpu.VMEM((CH, 16), jnp.float32)] * 2
            + [pltpu.VMEM((CH,), jnp.int32)] * 2
            + [pltpu.SemaphoreType.DMA] * 6
        ),
    )
    def k(pv_hbm, p_hbm, src_hbm, z_hbm, outacc_hbm, acc_sp,
          dbA, dbB, pbA, pbB, ixA, ixB, gA, gB, giA, giB, sA, sB):
        c = lax.axis_index("c")
        s = lax.axis_index("s")
        base = s * perc
        pltpu.sync_copy(z_hbm, acc_sp.at[pl.ds(s * TROWS, TROWS)])

        A = (dbA, pbA, ixA, gA, giA, sA)
        B = (dbB, pbB, ixB, gB, giB, sB)

        def issue_loads(j, bufs, core):
            db, pb, ix, g, gi, _ = bufs
            chunk = base + j
            pltpu.async_copy(src_hbm.at[dbase + chunk], ix, gi)
            if core == 0:
                pltpu.async_copy(pv_hbm.at[chunk], db, g)
            else:
                pltpu.async_copy(p_hbm.at[chunk], pb, g)

        def wait_loads(j, bufs, core):
            db, pb, ix, g, gi, _ = bufs
            chunk = base + j
            pltpu.make_async_copy(src_hbm.at[dbase + chunk], ix, gi).wait()
            if core == 0:
                pltpu.make_async_copy(pv_hbm.at[chunk], db, g).wait()
            else:
                pltpu.make_async_copy(p_hbm.at[chunk], pb, g).wait()

        def issue_scatter(bufs, core):
            db, pb, ix, _, _, ss = bufs
            if core == 1:
                def ug(r, carry2):
                    db[r, pl.ds(0, 16)] = pb[r, ...]
                    return carry2

                lax.fori_loop(0, CH, ug, 0)
            pltpu.async_copy(db, acc_sp.at[ix], ss, add=True)

        def wait_scatter(bufs):
            db, _, ix, _, _, ss = bufs
            pltpu.make_async_copy(db, acc_sp.at[ix], ss).wait()

        def core_loop(core):
            if core == 1:
                pltpu.sync_copy(z_hbm.at[pl.ds(0, CH)], dbA)
                pltpu.sync_copy(z_hbm.at[pl.ds(0, CH)], dbB)
            plsc.subcore_barrier()
            issue_loads(0, A, core)

            def step(j, cur, nxt):
                @pl.when(j + 1 < perc)
                def _():
                    @pl.when(j >= 1)
                    def _():
                        wait_scatter(nxt)

                    issue_loads(j + 1, nxt, core)

                wait_loads(j, cur, core)
                issue_scatter(cur, core)

            def body(j, carry):
                @pl.when(lax.rem(j, 2) == 0)
                def _():
                    step(j, A, B)

                @pl.when(lax.rem(j, 2) == 1)
                def _():
                    step(j, B, A)

                return carry

            lax.fori_loop(0, perc, body, 0)
            wait_scatter(A if (perc - 1) % 2 == 0 else B)
            wait_scatter(A if (perc - 2) % 2 == 0 else B)

        @pl.when(c == 0)
        def _():
            core_loop(0)

        @pl.when(c == 1)
        def _():
            core_loop(1)

        plsc.subcore_barrier()
        pltpu.sync_copy(acc_sp.at[pl.ds(s * TROWS, TROWS)],
                        outacc_hbm.at[c, pl.ds(s * TROWS, TROWS)])

    return k(pv3, p3, src2, z128)


def _ln(x, g, b):
    m = jnp.mean(x, axis=-1, keepdims=True)
    v = jnp.mean((x - m) ** 2, axis=-1, keepdims=True)
    return (x - m) * lax.rsqrt(v + 1e-5) * g + b


def _final_body(*refs):
    nacc = len(PART_NC)
    acc_refs = refs[:nacc]
    (hn_ref, ex_ref, wo_ref, bo_ref, g1_ref, bl1_ref,
     w1_ref, b1_ref, w2_ref, b2_ref, g2_ref, bl2_ref, out_ref) = refs[nacc:]
    acc = acc_refs[0][0]
    s16 = acc_refs[0][1][:, :16]
    for r in acc_refs[1:]:
        acc = acc + r[0]
        s16 = s16 + r[1][:, :16]
    den = jnp.dot(s16, ex_ref[...], preferred_element_type=jnp.float32)
    den = jnp.where(den == 0.0, 1.0, den)
    attn = acc / den
    o = jnp.dot(attn, wo_ref[...], preferred_element_type=jnp.float32) + bo_ref[...]
    x = hn_ref[...] + o
    h1 = _ln(x, g1_ref[...], bl1_ref[...])
    f = jnp.dot(h1, w1_ref[...], preferred_element_type=jnp.float32) + b1_ref[...]
    f = 0.5 * f * (1.0 + lax.erf(f * _INV_SQRT2))
    f = jnp.dot(f, w2_ref[...], preferred_element_type=jnp.float32) + b2_ref[...]
    out_ref[...] = _ln(h1 + f, g2_ref[...], bl2_ref[...])


def _final_stage(accs, h_n, ex16, wo_t, bo, g1, bl1, w1_t, b1_, w2_t, b2_, g2, bl2):
    blk = 2000
    full = lambda shape: pl.BlockSpec(shape, lambda i: tuple(0 for _ in shape))
    return pl.pallas_call(
        _final_body,
        grid=(N // blk,),
        in_specs=[
            pl.BlockSpec((2, blk, D), lambda i: (0, i, 0))
            for _ in PART_NC
        ] + [
            pl.BlockSpec((blk, D), lambda i: (i, 0)),
            full((16, D)),
            full((D, D)),
            full((1, D)),
            full((1, D)),
            full((1, D)),
            full((D, 4 * D)),
            full((1, 4 * D)),
            full((4 * D, D)),
            full((1, D)),
            full((1, D)),
            full((1, D)),
        ],
        out_specs=pl.BlockSpec((blk, D), lambda i: (i, 0)),
        out_shape=jax.ShapeDtypeStruct((N, D), jnp.float32),
    )(*accs, h_n, ex16, wo_t, bo, g1, bl1, w1_t, b1_, w2_t, b2_, g2, bl2)


def kernel(h_n, h_e, edge_index, Wq_w, Wq_b, Wkv_w, Wkv_b, Wo_w, Wo_b,
           ln1_g, ln1_b, W1, b1, W2, b2, ln2_g, ln2_b):
    f32 = jnp.float32
    src = edge_index[0]
    dst = edge_index[1]

    w_node = jnp.concatenate([Wq_w[:, :D].T, Wkv_w[:, :D].T], axis=1)
    b_node = jnp.concatenate([Wq_b, Wkv_b])[None, :]
    w_edge = jnp.concatenate([Wq_w[:, D:].T, Wkv_w[:, D:].T], axis=1)

    hd_ids = jnp.arange(D) // HD
    m16 = (hd_ids[:, None] == jnp.arange(16)[None, :]).astype(f32)
    ex16 = m16.T

    z128 = jnp.zeros((TROWS, D), f32)

    qn, kn, vn = _node_proj(h_n, w_node, b_node)

    src2 = src.reshape(NC, CH)
    dst2 = dst.reshape(NC, CH)

    accs = []
    hoff = 0
    for nchunk in PART_NC:
        qns, kns, vns = _sc_gather(qn, kn, vn, src2, dst2, nchunk, hoff)
        ne = nchunk * CH
        pv, p = _edge_stage(h_e, qns.reshape(ne, D), kns.reshape(ne, D),
                            vns.reshape(ne, D), w_edge, m16, ex16,
                            nchunk, hoff // CPB)
        accs.append(_sc_scatter(pv.reshape(nchunk, CH, D),
                                p.reshape(nchunk, CH, 16),
                                src2, z128, nchunk, hoff))
        hoff += nchunk

    return _final_stage(accs, h_n, ex16,
                        Wo_w.T, Wo_b[None, :], ln1_g[None, :], ln1_b[None, :],
                        W1.T, b1[None, :], W2.T, b2[None, :],
                        ln2_g[None, :], ln2_b[None, :])


# speedup vs baseline: 1.0748x; 1.0748x over previous
import functools
import math

import jax
import jax.numpy as jnp
from jax import lax
from jax.experimental import pallas as pl
from jax.experimental.pallas import tpu as pltpu
from jax.experimental.pallas import tpu_sc as plsc

N = 10000
E = 320000
D = 128
NH = 4
HD = 32

NW = 32
CH = 80
NC = E // CH
CPB = 32
PART_NC = (1984, 2016)

NPAD = 10240
TROWS = NPAD // 16

_INV_SQRT_HD = 1.0 / math.sqrt(HD)
_INV_SQRT2 = 1.0 / math.sqrt(2.0)


def _node_proj_body(hn_ref, w_ref, b_ref, qn_ref, kn_ref, vn_ref):
    o = jnp.dot(hn_ref[...], w_ref[...], preferred_element_type=jnp.float32)
    o = o + b_ref[...]
    qn_ref[...] = o[:, :D]
    kn_ref[...] = o[:, D:2 * D]
    vn_ref[...] = o[:, 2 * D:]


def _node_proj(h_n, w_node, b_node):
    blk = 2000
    return pl.pallas_call(
        _node_proj_body,
        grid=(N // blk,),
        in_specs=[
            pl.BlockSpec((blk, D), lambda i: (i, 0)),
            pl.BlockSpec((D, 3 * D), lambda i: (0, 0)),
            pl.BlockSpec((1, 3 * D), lambda i: (0, 0)),
        ],
        out_specs=[
            pl.BlockSpec((blk, D), lambda i: (i, 0)),
            pl.BlockSpec((blk, D), lambda i: (i, 0)),
            pl.BlockSpec((blk, D), lambda i: (i, 0)),
        ],
        out_shape=[
            jax.ShapeDtypeStruct((N, D), jnp.float32),
            jax.ShapeDtypeStruct((N, D), jnp.float32),
            jax.ShapeDtypeStruct((N, D), jnp.float32),
        ],
    )(h_n, w_node, b_node)


def _sc_gather(qn, kn, vn, src2, dst2, nchunk, hoff):
    perw = nchunk // NW
    mesh = plsc.VectorSubcoreMesh(core_axis_name="c", subcore_axis_name="s")

    @functools.partial(
        pl.kernel,
        out_type=(
            jax.ShapeDtypeStruct((nchunk, CH, D), jnp.float32),
            jax.ShapeDtypeStruct((nchunk, CH, D), jnp.float32),
            jax.ShapeDtypeStruct((nchunk, CH, D), jnp.float32),
        ),
        mesh=mesh,
        scratch_types=(
            [pltpu.VMEM((CH,), jnp.int32)] * 4
            + [pltpu.VMEM((CH, D), jnp.float32)] * 6
            + [pltpu.SemaphoreType.DMA] * 16
        ),
    )
    def k(qn_hbm, kn_hbm, vn_hbm, src_hbm, dst_hbm, qns_hbm, kns_hbm, vns_hbm,
          sxA, dxA, sxB, dxB, qbA, kbA, vbA, qbB, kbB, vbB,
          isA, idA, isB, idB,
          gqA, gkA, gvA, gqB, gkB, gvB, wqA, wkA, wvA, wqB, wkB, wvB):
        c = lax.axis_index("c")
        s = lax.axis_index("s")
        wid = s * 2 + c
        base = wid * perw

        A = (sxA, dxA, qbA, kbA, vbA, isA, idA, gqA, gkA, gvA, wqA, wkA, wvA)
        B = (sxB, dxB, qbB, kbB, vbB, isB, idB, gqB, gkB, gvB, wqB, wkB, wvB)

        def issue_ix(j, bufs):
            sx, dx = bufs[0], bufs[1]
            si, di = bufs[5], bufs[6]
            g = hoff + base + j
            pltpu.async_copy(src_hbm.at[g], sx, si)
            pltpu.async_copy(dst_hbm.at[g], dx, di)

        def wait_ix(j, bufs):
            sx, dx = bufs[0], bufs[1]
            si, di = bufs[5], bufs[6]
            g = hoff + base + j
            pltpu.make_async_copy(src_hbm.at[g], sx, si).wait()
            pltpu.make_async_copy(dst_hbm.at[g], dx, di).wait()

        def issue_g(j, bufs):
            sx, dx, qb, kb, vb = bufs[:5]
            gq, gk, gv = bufs[7:10]
            pltpu.async_copy(qn_hbm.at[sx], qb, gq)
            pltpu.async_copy(kn_hbm.at[dx], kb, gk)
            pltpu.async_copy(vn_hbm.at[dx], vb, gv)

        def wait_g(j, bufs):
            sx, dx, qb, kb, vb = bufs[:5]
            gq, gk, gv = bufs[7:10]
            pltpu.make_async_copy(qn_hbm.at[sx], qb, gq).wait()
            pltpu.make_async_copy(kn_hbm.at[dx], kb, gk).wait()
            pltpu.make_async_copy(vn_hbm.at[dx], vb, gv).wait()

        def issue_w(j, bufs):
            qb, kb, vb = bufs[2:5]
            wq, wk, wv = bufs[10:]
            chunk = base + j
            pltpu.async_copy(qb, qns_hbm.at[chunk], wq)
            pltpu.async_copy(kb, kns_hbm.at[chunk], wk)
            pltpu.async_copy(vb, vns_hbm.at[chunk], wv)

        def wait_w(j, bufs):
            qb, kb, vb = bufs[2:5]
            wq, wk, wv = bufs[10:]
            chunk = base + j
            pltpu.make_async_copy(qb, qns_hbm.at[chunk], wq).wait()
            pltpu.make_async_copy(kb, kns_hbm.at[chunk], wk).wait()
            pltpu.make_async_copy(vb, vns_hbm.at[chunk], wv).wait()

        def step(j, cur, nxt):
            @pl.when(j + 1 < perw)
            def _():
                @pl.when(j >= 1)
                def _():
                    wait_w(j - 1, nxt)

                wait_ix(j + 1, nxt)
                issue_g(j + 1, nxt)

            wait_g(j, cur)
            @pl.when(j + 2 < perw)
            def _():
                issue_ix(j + 2, cur)

            issue_w(j, cur)

        issue_ix(0, A)
        wait_ix(0, A)
        issue_g(0, A)

        @pl.when(1 < perw)
        def _():
            issue_ix(1, B)

        def body(j, carry):
            @pl.when(lax.rem(j, 2) == 0)
            def _():
                step(j, A, B)

            @pl.when(lax.rem(j, 2) == 1)
            def _():
                step(j, B, A)

            return carry

        lax.fori_loop(0, perw, body, 0)
        wait_w(perw - 1, A if (perw - 1) % 2 == 0 else B)
        wait_w(perw - 2, A if (perw - 2) % 2 == 0 else B)

    return k(qn, kn, vn, src2, dst2)


def _edge_body(he_ref, qns_ref, kns_ref, vns_ref, we_ref, m16_ref, ex_ref, pv_ref, p_ref):
    he = he_ref[...]
    qkve = jnp.dot(he, we_ref[...], preferred_element_type=jnp.float32)
    q = qkve[:, :D] + qns_ref[...]
    k = qkve[:, D:2 * D] + kns_ref[...]
    v = qkve[:, 2 * D:] + vns_ref[...]
    s16 = jnp.dot(q * k, m16_ref[...], preferred_element_type=jnp.float32)
    s16 = s16 * _INV_SQRT_HD
    col = lax.broadcasted_iota(jnp.int32, s16.shape, 1)
    p16 = jnp.where(col < NH, jnp.exp(s16), 0.0)
    pv_ref[...] = v * jnp.dot(p16, ex_ref[...], preferred_element_type=jnp.float32)
    p_ref[...] = p16


def _edge_stage(h_e, qns, kns, vns, w_edge, m16, ex16, nchunk, hblk):
    blk = CPB * CH
    ne = nchunk * CH
    return pl.pallas_call(
        _edge_body,
        grid=(ne // blk,),
        in_specs=[
            pl.BlockSpec((blk, D), lambda i: (i + hblk, 0)),
            pl.BlockSpec((blk, D), lambda i: (i, 0)),
            pl.BlockSpec((blk, D), lambda i: (i, 0)),
            pl.BlockSpec((blk, D), lambda i: (i, 0)),
            pl.BlockSpec((D, 3 * D), lambda i: (0, 0)),
            pl.BlockSpec((D, 16), lambda i: (0, 0)),
            pl.BlockSpec((16, D), lambda i: (0, 0)),
        ],
        out_specs=[
            pl.BlockSpec((blk, D), lambda i: (i, 0)),
            pl.BlockSpec((blk, 16), lambda i: (i, 0)),
        ],
        out_shape=[
            jax.ShapeDtypeStruct((ne, D), jnp.float32),
            jax.ShapeDtypeStruct((ne, 16), jnp.float32),
        ],
    )(h_e, qns, kns, vns, w_edge, m16, ex16)


def _sc_scatter(pv3, p3, src2, z128, nchunk, dbase):
    perc = nchunk // 16
    mesh = plsc.VectorSubcoreMesh(core_axis_name="c", subcore_axis_name="s")

    @functools.partial(
        pl.kernel,
        out_type=jax.ShapeDtypeStruct((2, NPAD, D), jnp.float32),
        mesh=mesh,
        scratch_types=(
            [pltpu.VMEM_SHARED((NPAD, D), jnp.float32)]
            + [pltpu.VMEM((CH, D), jnp.float32)] * 2
            + [pltpu.VMEM((CH, 16), jnp.float32)] * 2
            + [pltpu.VMEM((CH,), jnp.int32)] * 2
            + [pltpu.SemaphoreType.DMA] * 6
        ),
    )
    def k(pv_hbm, p_hbm, src_hbm, z_hbm, outacc_hbm, acc_sp,
          dbA, dbB, pbA, pbB, ixA, ixB, gA, gB, giA, giB, sA, sB):
        c = lax.axis_index("c")
        s = lax.axis_index("s")
        base = s * perc
        pltpu.sync_copy(z_hbm, acc_sp.at[pl.ds(s * TROWS, TROWS)])

        A = (dbA, pbA, ixA, gA, giA, sA)
        B = (dbB, pbB, ixB, gB, giB, sB)

        def issue_loads(j, bufs, core):
            db, pb, ix, g, gi, _ = bufs
            chunk = base + j
            pltpu.async_copy(src_hbm.at[dbase + chunk], ix, gi)
            if core == 0:
                pltpu.async_copy(pv_hbm.at[chunk], db, g)
            else:
                pltpu.async_copy(p_hbm.at[chunk], pb, g)

        def wait_loads(j, bufs, core):
            db, pb, ix, g, gi, _ = bufs
            chunk = base + j
            pltpu.make_async_copy(src_hbm.at[dbase + chunk], ix, gi).wait()
            if core == 0:
                pltpu.make_async_copy(pv_hbm.at[chunk], db, g).wait()
            else:
                pltpu.make_async_copy(p_hbm.at[chunk], pb, g).wait()

        def issue_scatter(bufs, core):
            db, pb, ix, _, _, ss = bufs
            if core == 1:
                def ug(r, carry2):
                    db[r, pl.ds(0, 16)] = pb[r, ...]
                    return carry2

                lax.fori_loop(0, CH, ug, 0)
            pltpu.async_copy(db, acc_sp.at[ix], ss, add=True)

        def wait_scatter(bufs):
            db, _, ix, _, _, ss = bufs
            pltpu.make_async_copy(db, acc_sp.at[ix], ss).wait()

        def core_loop(core):
            if core == 1:
                pltpu.sync_copy(z_hbm.at[pl.ds(0, CH)], dbA)
                pltpu.sync_copy(z_hbm.at[pl.ds(0, CH)], dbB)
            plsc.subcore_barrier()
            issue_loads(0, A, core)

            def step(j, cur, nxt):
                @pl.when(j + 1 < perc)
                def _():
                    @pl.when(j >= 1)
                    def _():
                        wait_scatter(nxt)

                    issue_loads(j + 1, nxt, core)

                wait_loads(j, cur, core)
                issue_scatter(cur, core)

            def body(j, carry):
                @pl.when(lax.rem(j, 2) == 0)
                def _():
                    step(j, A, B)

                @pl.when(lax.rem(j, 2) == 1)
                def _():
                    step(j, B, A)

                return carry

            lax.fori_loop(0, perc, body, 0)
            wait_scatter(A if (perc - 1) % 2 == 0 else B)
            wait_scatter(A if (perc - 2) % 2 == 0 else B)

        @pl.when(c == 0)
        def _():
            core_loop(0)

        @pl.when(c == 1)
        def _():
            core_loop(1)

        plsc.subcore_barrier()
        pltpu.sync_copy(acc_sp.at[pl.ds(s * TROWS, TROWS)],
                        outacc_hbm.at[c, pl.ds(s * TROWS, TROWS)])

    return k(pv3, p3, src2, z128)


def _ln(x, g, b):
    m = jnp.mean(x, axis=-1, keepdims=True)
    v = jnp.mean((x - m) ** 2, axis=-1, keepdims=True)
    return (x - m) * lax.rsqrt(v + 1e-5) * g + b


def _final_body(*refs):
    nacc = len(PART_NC)
    acc_refs = refs[:nacc]
    (hn_ref, ex_ref, wo_ref, bo_ref, g1_ref, bl1_ref,
     w1_ref, b1_ref, w2_ref, b2_ref, g2_ref, bl2_ref, out_ref) = refs[nacc:]
    acc = acc_refs[0][0]
    s16 = acc_refs[0][1][:, :16]
    for r in acc_refs[1:]:
        acc = acc + r[0]
        s16 = s16 + r[1][:, :16]
    den = jnp.dot(s16, ex_ref[...], preferred_element_type=jnp.float32)
    den = jnp.where(den == 0.0, 1.0, den)
    attn = acc / den
    o = jnp.dot(attn, wo_ref[...], preferred_element_type=jnp.float32) + bo_ref[...]
    x = hn_ref[...] + o
    h1 = _ln(x, g1_ref[...], bl1_ref[...])
    f = jnp.dot(h1, w1_ref[...], preferred_element_type=jnp.float32) + b1_ref[...]
    f = 0.5 * f * (1.0 + lax.erf(f * _INV_SQRT2))
    f = jnp.dot(f, w2_ref[...], preferred_element_type=jnp.float32) + b2_ref[...]
    out_ref[...] = _ln(h1 + f, g2_ref[...], bl2_ref[...])


def _final_stage(accs, h_n, ex16, wo_t, bo, g1, bl1, w1_t, b1_, w2_t, b2_, g2, bl2):
    blk = 2000
    full = lambda shape: pl.BlockSpec(shape, lambda i: tuple(0 for _ in shape))
    return pl.pallas_call(
        _final_body,
        grid=(N // blk,),
        in_specs=[
            pl.BlockSpec((2, blk, D), lambda i: (0, i, 0))
            for _ in PART_NC
        ] + [
            pl.BlockSpec((blk, D), lambda i: (i, 0)),
            full((16, D)),
            full((D, D)),
            full((1, D)),
            full((1, D)),
            full((1, D)),
            full((D, 4 * D)),
            full((1, 4 * D)),
            full((4 * D, D)),
            full((1, D)),
            full((1, D)),
            full((1, D)),
        ],
        out_specs=pl.BlockSpec((blk, D), lambda i: (i, 0)),
        out_shape=jax.ShapeDtypeStruct((N, D), jnp.float32),
    )(*accs, h_n, ex16, wo_t, bo, g1, bl1, w1_t, b1_, w2_t, b2_, g2, bl2)


def kernel(h_n, h_e, edge_index, Wq_w, Wq_b, Wkv_w, Wkv_b, Wo_w, Wo_b,
           ln1_g, ln1_b, W1, b1, W2, b2, ln2_g, ln2_b):
    f32 = jnp.float32
    src = edge_index[0]
    dst = edge_index[1]

    w_node = jnp.concatenate([Wq_w[:, :D].T, Wkv_w[:, :D].T], axis=1)
    b_node = jnp.concatenate([Wq_b, Wkv_b])[None, :]
    w_edge = jnp.concatenate([Wq_w[:, D:].T, Wkv_w[:, D:].T], axis=1)

    hd_ids = jnp.arange(D) // HD
    m16 = (hd_ids[:, None] == jnp.arange(16)[None, :]).astype(f32)
    ex16 = m16.T

    z128 = jnp.zeros((TROWS, D), f32)

    qn, kn, vn = _node_proj(h_n, w_node, b_node)

    src2 = src.reshape(NC, CH)
    dst2 = dst.reshape(NC, CH)

    accs = []
    hoff = 0
    for nchunk in PART_NC:
        qns, kns, vns = _sc_gather(qn, kn, vn, src2, dst2, nchunk, hoff)
        ne = nchunk * CH
        pv, p = _edge_stage(h_e, qns.reshape(ne, D), kns.reshape(ne, D),
                            vns.reshape(ne, D), w_edge, m16, ex16,
                            nchunk, hoff // CPB)
        accs.append(_sc_scatter(pv.reshape(nchunk, CH, D),
                                p.reshape(nchunk, CH, 16),
                                src2, z128, nchunk, hoff))
        hoff += nchunk

    return _final_stage(accs, h_n, ex16,
                        Wo_w.T, Wo_b[None, :], ln1_g[None, :], ln1_b[None, :],
                        W1.T, b1[None, :], W2.T, b2[None, :],
                        ln2_g[None, :], ln2_b[None, :])
